# Initial kernel scaffold; baseline (speedup 1.0000x reference)
import jax
import jax.numpy as jnp
from jax.experimental import pallas as pl


def kernel(x_raw, field_features, table, w):
    raise NotImplementedError("write your pallas kernel here")


# speedup vs baseline: 59.6347x; 59.6347x over previous
import functools

import jax
import jax.numpy as jnp
from jax.experimental import pallas as pl
from jax.experimental.pallas import tpu as pltpu

_BAND = 128


def _round_up(x, m):
    return (x + m - 1) // m * m


def _fm_kernel(idx_t_ref, lhs_t_ref, out_ref, counts_ref, *, emb_dim, n_fields):
    _, n_blk = idx_t_ref.shape
    d_pad, _ = lhs_t_ref.shape

    idx_all = idx_t_ref[...]
    iota_b = jax.lax.broadcasted_iota(jnp.int32, (_BAND, n_blk), 0)

    for f in range(n_fields):
        hit = (iota_b == idx_all[f : f + 1, :]).astype(jnp.float32)
        counts_ref[f * _BAND : (f + 1) * _BAND, :] = hit

    res = jnp.dot(lhs_t_ref[...], counts_ref[...],
                  preferred_element_type=jnp.float32)

    row_id = jax.lax.broadcasted_iota(jnp.int32, (d_pad, n_blk), 0)
    contrib = jnp.where(row_id < emb_dim, res * res, res)
    logit = jnp.sum(contrib, axis=0, keepdims=True)
    out_ref[...] = jax.nn.sigmoid(logit)


def _fm_forward(x_raw, field_features, table, w, *, n_blk=256):
    N, F = x_raw.shape
    V, D = table.shape

    n_pad = _round_up(N, n_blk)
    d_rows = D + 1
    d_pad = _round_up(d_rows, 8)
    v_loc = F * _BAND

    idx_t = jnp.pad(x_raw.astype(jnp.int32), ((0, n_pad - N), (0, 0))).T

    offsets = jnp.concatenate(
        [jnp.zeros((1,), jnp.int32),
         jnp.cumsum(jnp.asarray(field_features, jnp.int32))[:-1]])
    q = w[:, 0] - jnp.sum(table * table, axis=1)
    fused = jnp.concatenate([table, q[:, None]], axis=1)
    j = jnp.arange(_BAND, dtype=jnp.int32)
    gcol = offsets[:, None] + j[None, :]
    valid = (j[None, :] < jnp.asarray(field_features, jnp.int32)[:, None])
    cols = jnp.where(valid, gcol, 0).reshape(-1)
    lhs_t = (fused.T[:, cols] * valid.reshape(-1)[None, :].astype(jnp.float32))
    lhs_t = jnp.pad(lhs_t, ((0, d_pad - d_rows), (0, 0)))

    vmem_bytes = int(
        4 * (2 * d_pad * v_loc
             + 2 * F * n_blk
             + 2 * 1 * n_blk
             + v_loc * n_blk
             + 2 * d_pad * n_blk)
        + (8 << 20))

    out = pl.pallas_call(
        functools.partial(_fm_kernel, emb_dim=D, n_fields=F),
        out_shape=jax.ShapeDtypeStruct((1, n_pad), jnp.float32),
        grid=(n_pad // n_blk,),
        in_specs=[
            pl.BlockSpec((F, n_blk), lambda i: (0, i)),
            pl.BlockSpec((d_pad, v_loc), lambda i: (0, 0)),
        ],
        out_specs=pl.BlockSpec((1, n_blk), lambda i: (0, i)),
        scratch_shapes=[pltpu.VMEM((v_loc, n_blk), jnp.float32)],
        compiler_params=pltpu.CompilerParams(
            dimension_semantics=("parallel",),
            vmem_limit_bytes=vmem_bytes),
    )(idx_t, lhs_t)

    return out[0, :N].reshape(N, 1)


def kernel(x_raw, field_features, table, w):
    return _fm_forward(x_raw, field_features, table, w)


# speedup vs baseline: 63.6856x; 1.0679x over previous
import functools

import jax
import jax.numpy as jnp
from jax.experimental import pallas as pl
from jax.experimental.pallas import tpu as pltpu

_BAND = 128


def _round_up(x, m):
    return (x + m - 1) // m * m


def _fm_kernel(idx_t_ref, lhs_t_ref, out_ref, counts_ref, *, emb_dim, n_fields):
    _, n_blk = idx_t_ref.shape
    d_pad, _ = lhs_t_ref.shape

    idx_all = idx_t_ref[...]
    iota_b = jax.lax.broadcasted_iota(jnp.int32, (_BAND, n_blk), 0)

    for f in range(n_fields):
        hit = (iota_b == idx_all[f : f + 1, :]).astype(jnp.float32)
        counts_ref[f * _BAND : (f + 1) * _BAND, :] = hit

    res = jnp.dot(lhs_t_ref[...], counts_ref[...],
                  preferred_element_type=jnp.float32)

    row_id = jax.lax.broadcasted_iota(jnp.int32, (d_pad, n_blk), 0)
    contrib = jnp.where(row_id < emb_dim, res * res, res)
    logit = jnp.sum(contrib, axis=0, keepdims=True)
    out_ref[...] = jax.nn.sigmoid(logit)


def _fm_forward(x_raw, field_features, table, w, *, n_blk=512):
    N, F = x_raw.shape
    V, D = table.shape

    n_pad = _round_up(N, n_blk)
    d_rows = D + 1
    d_pad = _round_up(d_rows, 8)
    v_loc = F * _BAND

    idx_t = jnp.pad(x_raw.astype(jnp.int32), ((0, n_pad - N), (0, 0))).T

    offsets = jnp.concatenate(
        [jnp.zeros((1,), jnp.int32),
         jnp.cumsum(jnp.asarray(field_features, jnp.int32))[:-1]])
    q = w[:, 0] - jnp.sum(table * table, axis=1)
    fused_t = jnp.concatenate([table, q[:, None]], axis=1).T
    fused_t = jnp.pad(fused_t, ((0, 0), (0, _BAND)))
    j = jnp.arange(_BAND, dtype=jnp.int32)
    valid = (j[None, :] < jnp.asarray(field_features, jnp.int32)[:, None])
    bands = [jax.lax.dynamic_slice_in_dim(fused_t, offsets[f], _BAND, axis=1)
             for f in range(F)]
    lhs_t = jnp.concatenate(bands, axis=1) * valid.reshape(-1)[None, :]
    lhs_t = jnp.pad(lhs_t, ((0, d_pad - d_rows), (0, 0)))

    vmem_bytes = int(
        4 * (2 * d_pad * v_loc
             + 2 * F * n_blk
             + 2 * 1 * n_blk
             + v_loc * n_blk
             + 2 * d_pad * n_blk)
        + (8 << 20))

    out = pl.pallas_call(
        functools.partial(_fm_kernel, emb_dim=D, n_fields=F),
        out_shape=jax.ShapeDtypeStruct((1, n_pad), jnp.float32),
        grid=(n_pad // n_blk,),
        in_specs=[
            pl.BlockSpec((F, n_blk), lambda i: (0, i)),
            pl.BlockSpec((d_pad, v_loc), lambda i: (0, 0)),
        ],
        out_specs=pl.BlockSpec((1, n_blk), lambda i: (0, i)),
        scratch_shapes=[pltpu.VMEM((v_loc, n_blk), jnp.float32)],
        compiler_params=pltpu.CompilerParams(
            dimension_semantics=("parallel",),
            vmem_limit_bytes=vmem_bytes),
    )(idx_t, lhs_t)

    return out[0, :N].reshape(N, 1)


def kernel(x_raw, field_features, table, w):
    return _fm_forward(x_raw, field_features, table, w)
